# Initial kernel scaffold; baseline (speedup 1.0000x reference)
import jax
import jax.numpy as jnp
from jax.experimental import pallas as pl


def kernel(init_emb, W1, b1, W2, b2, edge_index):
    raise NotImplementedError("write your pallas kernel here")


# speedup vs baseline: 3.1717x; 3.1717x over previous
import functools

import jax
import jax.numpy as jnp
from jax import lax
from jax.experimental import pallas as pl
from jax.experimental.pallas import tpu as pltpu
from jax.experimental.pallas import tpu_sc as plsc

NCORE = 2
NS = 16
LANES = 16
EB = 80
RBLK = 1000

_SC_PARAMS = pltpu.CompilerParams(
    needs_layout_passes=False, use_tc_tiling_on_sc=False)


def _degree_kernel(N, E):
    per_tec = E // NS
    n_vec = per_tec // LANES
    mesh = plsc.VectorSubcoreMesh(core_axis_name="c", subcore_axis_name="s")

    @functools.partial(
        pl.kernel,
        mesh=mesh,
        out_type=jax.ShapeDtypeStruct((NCORE, NS, N), jnp.float32),
        compiler_params=_SC_PARAMS,
        scratch_types=[
            pltpu.VMEM((per_tec,), jnp.int32),
            pltpu.VMEM((N,), jnp.float32),
        ],
    )
    def deg_k(ei, out, ibuf, hist):
        cid = lax.axis_index("c")
        sid = lax.axis_index("s")
        pltpu.sync_copy(ei.at[cid].at[sid], ibuf)

        z16 = jnp.zeros((LANES,), jnp.float32)

        def zero_body(i, _):
            hist[pl.ds(i * LANES, LANES)] = z16
            return 0

        lax.fori_loop(0, N // LANES, zero_body, 0)

        ones16 = jnp.ones((LANES,), jnp.float32)

        def hist_body(i, _):
            idx = ibuf[pl.ds(i * LANES, LANES)]
            plsc.addupdate_scatter(hist, [idx], ones16)
            return 0

        lax.fori_loop(0, n_vec, hist_body, 0)
        pltpu.sync_copy(hist, out.at[cid].at[sid])

    return deg_k


def _spmm_kernel(N, E, C, Dc):
    per_tec = E // NS
    nb = per_tec // EB
    cpc = C // NCORE
    rows_per_tile = N // NS
    ZR = 25
    nzc = rows_per_tile // ZR
    mesh = plsc.VectorSubcoreMesh(core_axis_name="c", subcore_axis_name="s")

    @functools.partial(
        pl.kernel,
        mesh=mesh,
        out_type=jax.ShapeDtypeStruct((C, N, Dc), jnp.float32),
        compiler_params=_SC_PARAMS,
        scratch_types=[
            pltpu.VMEM((2, EB), jnp.int32),
            pltpu.VMEM((EB,), jnp.int32),
            pltpu.VMEM((EB, Dc), jnp.float32),
            pltpu.VMEM((ZR, Dc), jnp.float32),
            pltpu.VMEM_SHARED((N, Dc), jnp.float32),
            pltpu.SemaphoreType.DMA,
        ],
    )
    def spmm_k(tab, em, out, ebuf, idxadj, rows, zrow, acc, sem):
        cid = lax.axis_index("c")
        sid = lax.axis_index("s")

        z16 = jnp.zeros((LANES,), jnp.float32)

        def zfill(r, _):
            for t in range(Dc // LANES):
                zrow[r, pl.ds(t * LANES, LANES)] = z16
            return 0

        lax.fori_loop(0, ZR, zfill, 0)

        row0 = sid * rows_per_tile

        for ci in range(cpc):
            k = cid + NCORE * ci
            for j in range(nzc):
                pltpu.sync_copy(zrow, acc.at[pl.ds(row0 + j * ZR, ZR)])
            plsc.subcore_barrier()
            koff = k * N

            def ebody(i, _):
                pltpu.sync_copy(em.at[sid].at[i], ebuf)
                for t in range(EB // LANES):
                    sl = pl.ds(t * LANES, LANES)
                    idxadj[sl] = ebuf[0, sl] + koff
                pltpu.async_copy(tab.at[idxadj], rows, sem).wait()
                pltpu.sync_copy(rows, acc.at[ebuf.at[1]], add=True)
                return 0

            lax.fori_loop(0, nb, ebody, 0)
            plsc.subcore_barrier()
            for j in range(nzc):
                sl = pl.ds(row0 + j * ZR, ZR)
                pltpu.sync_copy(acc.at[sl], out.at[k].at[sl])

    return spmm_k


def _prescale(dp_t, emb, N):
    nblk = N // RBLK

    def body(dp_ref, emb_ref, h1_ref, nrm_ref):
        deg = jnp.sum(dp_ref[...], axis=2)
        nrm = jnp.where(deg > 0, lax.rsqrt(deg), 0.0)
        nrm_ref[...] = nrm
        h = emb_ref[...] * nrm[:, 0:1]
        h1_ref[0] = h[:, :160]
        h1_ref[1] = jnp.concatenate(
            [h[:, 160:300], jnp.zeros((RBLK, 20), jnp.float32)], axis=1)

    return pl.pallas_call(
        body,
        grid=(nblk,),
        in_specs=[
            pl.BlockSpec((RBLK, NCORE, NS), lambda i: (i, 0, 0)),
            pl.BlockSpec((RBLK, 300), lambda i: (i, 0)),
        ],
        out_specs=[
            pl.BlockSpec((2, RBLK, 160), lambda i: (0, i, 0)),
            pl.BlockSpec((RBLK, 2), lambda i: (i, 0)),
        ],
        out_shape=[
            jax.ShapeDtypeStruct((2, N, 160), jnp.float32),
            jax.ShapeDtypeStruct((N, 2), jnp.float32),
        ],
    )(dp_t, emb)


def _layer1(agg1, nrm, W1p, b1, N):
    nblk = N // RBLK

    def body(a_ref, n_ref, w_ref, b_ref, o_ref):
        x = jnp.concatenate([a_ref[0], a_ref[1]], axis=1)
        nb2 = n_ref[...]
        x = x * nb2[:, 1:2]
        y = jnp.dot(x, w_ref[...], preferred_element_type=jnp.float32)
        y = y + b_ref[...]
        y = jnp.where(y >= 0.0, y, 0.2 * y)
        y = y * nb2[:, 0:1]
        for kk in range(3):
            o_ref[kk] = y[:, 112 * kk:112 * (kk + 1)]
        o_ref[3] = jnp.concatenate(
            [y[:, 336:400], jnp.zeros((RBLK, 48), jnp.float32)], axis=1)

    return pl.pallas_call(
        body,
        grid=(nblk,),
        in_specs=[
            pl.BlockSpec((2, RBLK, 160), lambda i: (0, i, 0)),
            pl.BlockSpec((RBLK, 2), lambda i: (i, 0)),
            pl.BlockSpec((320, 400), lambda i: (0, 0)),
            pl.BlockSpec((1, 400), lambda i: (0, 0)),
        ],
        out_specs=pl.BlockSpec((4, RBLK, 112), lambda i: (0, i, 0)),
        out_shape=jax.ShapeDtypeStruct((4, N, 112), jnp.float32),
    )(agg1, nrm, W1p, b1)


def _layer2(agg2, nrm, W2p, b2, N):
    nblk = N // RBLK

    def body(a_ref, n_ref, w_ref, b_ref, o_ref):
        x = jnp.concatenate([a_ref[kk] for kk in range(4)], axis=1)
        x = x * n_ref[:, 1:2]
        o_ref[...] = (
            jnp.dot(x, w_ref[...], preferred_element_type=jnp.float32)
            + b_ref[...])

    return pl.pallas_call(
        body,
        grid=(nblk,),
        in_specs=[
            pl.BlockSpec((4, RBLK, 112), lambda i: (0, i, 0)),
            pl.BlockSpec((RBLK, 2), lambda i: (i, 0)),
            pl.BlockSpec((448, 512), lambda i: (0, 0)),
            pl.BlockSpec((1, 512), lambda i: (0, 0)),
        ],
        out_specs=pl.BlockSpec((RBLK, 512), lambda i: (i, 0)),
        out_shape=jax.ShapeDtypeStruct((N, 512), jnp.float32),
    )(agg2, nrm, W2p, b2)


def kernel(init_emb, W1, b1, W2, b2, edge_index):
    N = init_emb.shape[0]
    E = edge_index.shape[1]

    per_tec = E // NS
    nb = per_tec // EB
    ei3d = edge_index.reshape(2, NS, per_tec)
    e4d = jnp.stack(
        [edge_index[0].reshape(NS, nb, EB),
         edge_index[1].reshape(NS, nb, EB)], axis=2)

    W1p = jnp.concatenate([W1, jnp.zeros((20, 400), jnp.float32)], axis=0)
    W2p = jnp.concatenate([W2, jnp.zeros((48, 512), jnp.float32)], axis=0)
    b1r = b1.reshape(1, 400)
    b2r = b2.reshape(1, 512)

    deg_parts = _degree_kernel(N, E)(ei3d)
    dp_t = jnp.transpose(deg_parts, (2, 0, 1))

    h1, nrm = _prescale(dp_t, init_emb, N)
    agg1 = _spmm_kernel(N, E, 2, 160)(
        h1.reshape(2 * N, 160), e4d)

    h2 = _layer1(agg1, nrm, W1p, b1r, N)
    agg2 = _spmm_kernel(N, E, 4, 112)(
        h2.reshape(4 * N, 112), e4d)

    return _layer2(agg2, nrm, W2p, b2r, N)


# speedup vs baseline: 5.1246x; 1.6157x over previous
import functools

import jax
import jax.numpy as jnp
from jax import lax
from jax.experimental import pallas as pl
from jax.experimental.pallas import tpu as pltpu
from jax.experimental.pallas import tpu_sc as plsc

NCORE = 2
NS = 16
LANES = 16
EB = 80
RBLK = 1000

_SC_PARAMS = pltpu.CompilerParams(
    needs_layout_passes=False, use_tc_tiling_on_sc=False)


def _degree_kernel(N, E):
    per_tec = E // NS
    n_vec = per_tec // LANES
    mesh = plsc.VectorSubcoreMesh(core_axis_name="c", subcore_axis_name="s")

    @functools.partial(
        pl.kernel,
        mesh=mesh,
        out_type=jax.ShapeDtypeStruct((NCORE, NS, N), jnp.float32),
        compiler_params=_SC_PARAMS,
        scratch_types=[
            pltpu.VMEM((per_tec,), jnp.int32),
            pltpu.VMEM((N,), jnp.float32),
        ],
    )
    def deg_k(ei, out, ibuf, hist):
        cid = lax.axis_index("c")
        sid = lax.axis_index("s")
        pltpu.sync_copy(ei.at[cid].at[sid], ibuf)

        z16 = jnp.zeros((LANES,), jnp.float32)

        def zero_body(i, _):
            hist[pl.ds(i * LANES, LANES)] = z16
            return 0

        lax.fori_loop(0, N // LANES, zero_body, 0)

        ones16 = jnp.ones((LANES,), jnp.float32)

        def hist_body(i, _):
            idx = ibuf[pl.ds(i * LANES, LANES)]
            plsc.addupdate_scatter(hist, [idx], ones16)
            return 0

        lax.fori_loop(0, n_vec, hist_body, 0)
        pltpu.sync_copy(hist, out.at[cid].at[sid])

    return deg_k


def _spmm_kernel(N, E, C, Dc):
    per_tec = E // NS
    nb = per_tec // EB
    cpc = C // NCORE
    rows_per_tile = N // NS
    ZR = 25
    nzc = rows_per_tile // ZR
    mesh = plsc.VectorSubcoreMesh(core_axis_name="c", subcore_axis_name="s")

    @functools.partial(
        pl.kernel,
        mesh=mesh,
        out_type=jax.ShapeDtypeStruct((C, N, Dc), jnp.float32),
        compiler_params=_SC_PARAMS,
        scratch_types=[
            pltpu.VMEM((2, EB), jnp.int32),
            pltpu.VMEM((2, EB), jnp.int32),
            pltpu.VMEM((EB, Dc), jnp.float32),
            pltpu.VMEM((EB, Dc), jnp.float32),
            pltpu.VMEM((ZR, Dc), jnp.float32),
            pltpu.VMEM_SHARED((N, Dc), jnp.float32),
            pltpu.SemaphoreType.DMA,
            pltpu.SemaphoreType.DMA,
        ],
    )
    def spmm_k(tab, em, out, ebuf0, ebuf1, rows0, rows1, zrow, acc, s0, s1):
        cid = lax.axis_index("c")
        sid = lax.axis_index("s")
        ebufs = (ebuf0, ebuf1)
        rowss = (rows0, rows1)
        sems = (s0, s1)

        z16 = jnp.zeros((LANES,), jnp.float32)

        def zfill(r, _):
            for t in range(Dc // LANES):
                zrow[r, pl.ds(t * LANES, LANES)] = z16
            return 0

        lax.fori_loop(0, ZR, zfill, 0)

        row0 = sid * rows_per_tile

        for ci in range(cpc):
            k = cid + NCORE * ci
            for j in range(nzc):
                pltpu.sync_copy(zrow, acc.at[pl.ds(row0 + j * ZR, ZR)])
            plsc.subcore_barrier()
            emk = em.at[k].at[sid]

            for b in range(2):
                pltpu.sync_copy(emk.at[b], ebufs[b])
                pltpu.async_copy(tab.at[ebufs[b].at[0]], rowss[b], sems[b])

            def gbody(g, _):
                for b in range(2):
                    pltpu.make_async_copy(
                        tab.at[ebufs[b].at[0]], rowss[b], sems[b]).wait()
                    pltpu.sync_copy(rowss[b], acc.at[ebufs[b].at[1]], add=True)
                    pltpu.sync_copy(emk.at[2 * g + b + 2], ebufs[b])
                    pltpu.async_copy(
                        tab.at[ebufs[b].at[0]], rowss[b], sems[b])
                return 0

            lax.fori_loop(0, (nb - 2) // 2, gbody, 0)
            for b in range(2):
                pltpu.make_async_copy(
                    tab.at[ebufs[b].at[0]], rowss[b], sems[b]).wait()
                pltpu.sync_copy(rowss[b], acc.at[ebufs[b].at[1]], add=True)

            plsc.subcore_barrier()
            for j in range(nzc):
                sl = pl.ds(row0 + j * ZR, ZR)
                pltpu.sync_copy(acc.at[sl], out.at[k].at[sl])

    return spmm_k


def _prescale(dp_t, emb, N):
    nblk = N // RBLK

    def body(dp_ref, emb_ref, h1_ref, nrm_ref):
        deg = jnp.sum(dp_ref[...], axis=2)
        nrm = jnp.where(deg > 0, lax.rsqrt(deg), 0.0)
        nrm_ref[...] = nrm
        h = emb_ref[...] * nrm[:, 0:1]
        h1_ref[0] = h[:, :160]
        h1_ref[1] = jnp.concatenate(
            [h[:, 160:300], jnp.zeros((RBLK, 20), jnp.float32)], axis=1)

    return pl.pallas_call(
        body,
        grid=(nblk,),
        in_specs=[
            pl.BlockSpec((RBLK, NCORE, NS), lambda i: (i, 0, 0)),
            pl.BlockSpec((RBLK, 300), lambda i: (i, 0)),
        ],
        out_specs=[
            pl.BlockSpec((2, RBLK, 160), lambda i: (0, i, 0)),
            pl.BlockSpec((RBLK, 2), lambda i: (i, 0)),
        ],
        out_shape=[
            jax.ShapeDtypeStruct((2, N, 160), jnp.float32),
            jax.ShapeDtypeStruct((N, 2), jnp.float32),
        ],
    )(dp_t, emb)


def _layer1(agg1, nrm, W1p, b1, N):
    nblk = N // RBLK

    def body(a_ref, n_ref, w_ref, b_ref, o_ref):
        x = jnp.concatenate([a_ref[0], a_ref[1]], axis=1)
        nb2 = n_ref[...]
        x = x * nb2[:, 1:2]
        y = jnp.dot(x, w_ref[...], preferred_element_type=jnp.float32)
        y = y + b_ref[...]
        y = jnp.where(y >= 0.0, y, 0.2 * y)
        y = y * nb2[:, 0:1]
        for kk in range(3):
            o_ref[kk] = y[:, 112 * kk:112 * (kk + 1)]
        o_ref[3] = jnp.concatenate(
            [y[:, 336:400], jnp.zeros((RBLK, 48), jnp.float32)], axis=1)

    return pl.pallas_call(
        body,
        grid=(nblk,),
        in_specs=[
            pl.BlockSpec((2, RBLK, 160), lambda i: (0, i, 0)),
            pl.BlockSpec((RBLK, 2), lambda i: (i, 0)),
            pl.BlockSpec((320, 400), lambda i: (0, 0)),
            pl.BlockSpec((1, 400), lambda i: (0, 0)),
        ],
        out_specs=pl.BlockSpec((4, RBLK, 112), lambda i: (0, i, 0)),
        out_shape=jax.ShapeDtypeStruct((4, N, 112), jnp.float32),
    )(agg1, nrm, W1p, b1)


def _layer2(agg2, nrm, W2p, b2, N):
    nblk = N // RBLK

    def body(a_ref, n_ref, w_ref, b_ref, o_ref):
        x = jnp.concatenate([a_ref[kk] for kk in range(4)], axis=1)
        x = x * n_ref[:, 1:2]
        o_ref[...] = (
            jnp.dot(x, w_ref[...], preferred_element_type=jnp.float32)
            + b_ref[...])

    return pl.pallas_call(
        body,
        grid=(nblk,),
        in_specs=[
            pl.BlockSpec((4, RBLK, 112), lambda i: (0, i, 0)),
            pl.BlockSpec((RBLK, 2), lambda i: (i, 0)),
            pl.BlockSpec((448, 512), lambda i: (0, 0)),
            pl.BlockSpec((1, 512), lambda i: (0, 0)),
        ],
        out_specs=pl.BlockSpec((RBLK, 512), lambda i: (i, 0)),
        out_shape=jax.ShapeDtypeStruct((N, 512), jnp.float32),
    )(agg2, nrm, W2p, b2)


def kernel(init_emb, W1, b1, W2, b2, edge_index):
    N = init_emb.shape[0]
    E = edge_index.shape[1]

    per_tec = E // NS
    nb = per_tec // EB
    ei3d = edge_index.reshape(2, NS, per_tec)

    def edge_tables(C):
        src = edge_index[0].reshape(1, NS, nb, EB)
        dst = edge_index[1].reshape(1, NS, nb, EB)
        offs = (jnp.arange(C, dtype=jnp.int32) * N).reshape(C, 1, 1, 1)
        return jnp.stack(
            [src + offs, jnp.broadcast_to(dst, (C, NS, nb, EB))], axis=3)

    W1p = jnp.concatenate([W1, jnp.zeros((20, 400), jnp.float32)], axis=0)
    W2p = jnp.concatenate([W2, jnp.zeros((48, 512), jnp.float32)], axis=0)
    b1r = b1.reshape(1, 400)
    b2r = b2.reshape(1, 512)

    deg_parts = _degree_kernel(N, E)(ei3d)
    dp_t = jnp.transpose(deg_parts, (2, 0, 1))

    h1, nrm = _prescale(dp_t, init_emb, N)
    agg1 = _spmm_kernel(N, E, 2, 160)(
        h1.reshape(2 * N, 160), edge_tables(2))

    h2 = _layer1(agg1, nrm, W1p, b1r, N)
    agg2 = _spmm_kernel(N, E, 4, 112)(
        h2.reshape(4 * N, 112), edge_tables(4))

    return _layer2(agg2, nrm, W2p, b2r, N)
